# baseline (speedup 1.0000x reference)
import functools

import jax
import jax.numpy as jnp
from jax import lax
from jax.experimental import pallas as pl
from jax.experimental.pallas import tpu as pltpu
from jax.experimental.pallas import tpu_sc as plsc

C = 10000
D = 128
L = 16
NC = 2
NS = 16
NW = NC * NS
B = 96
NBUF = 4
N0 = 84
N1 = 132
STRIPE = 624
TAIL = C - NS * STRIPE


def _spmm_body(edges_hbm, vals_hbm, h_hbm, out_hbm,
               edge, ev, gat, acc,
               sg0, sg1, sg2, sg3, ss0, ss1, ss2, ss3,
               se0, se1, se2, se3):
    c = lax.axis_index("c")
    s = lax.axis_index("s")
    wid = c * NS + s
    nt = jnp.where(c == 0, N0, N1)
    sgs = [sg0, sg1, sg2, sg3]
    sss = [ss0, ss1, ss2, ss3]
    ses = [se0, se1, se2, se3]

    for b in range(NBUF):
        pltpu.async_copy(edges_hbm.at[wid, b], edge.at[b], ses[b])
        pltpu.async_copy(vals_hbm.at[wid, b], ev.at[b], ses[b])

    zero = jnp.zeros((L,), jnp.float32)

    def _zero_row(i, _):
        for h in range(D // L):
            gat[0, i, pl.ds(h * L, L)] = zero
        return 0

    lax.fori_loop(0, B, _zero_row, 0)
    for k in range(6):
        pltpu.sync_copy(gat.at[0], acc.at[pl.ds(s * STRIPE + k * B, B)])
    pltpu.sync_copy(gat.at[0, pl.ds(0, 48)],
                    acc.at[pl.ds(s * STRIPE + 6 * B, 48)])

    @pl.when(s == NS - 1)
    def _():
        pltpu.sync_copy(gat.at[0, pl.ds(0, TAIL)],
                        acc.at[pl.ds(NS * STRIPE, TAIL)])

    plsc.subcore_barrier()

    for b in range(2):
        pltpu.make_async_copy(edges_hbm.at[wid, b], edge.at[b],
                              ses[b]).wait()
        pltpu.make_async_copy(vals_hbm.at[wid, b], ev.at[b], ses[b]).wait()
        pltpu.async_copy(h_hbm.at[edge.at[b, 1]], gat.at[b], sgs[b])

    def _quad(kk, _):
        for b in range(NBUF):
            k = kk * NBUF + b
            b2 = (b + 2) % NBUF

            @pl.when(k >= 2)
            def _():
                pltpu.make_async_copy(
                    gat.at[b2], acc.at[edge.at[b2, 0]], sss[b2]).wait()

            @pl.when(k + 2 < nt)
            def _():
                pltpu.make_async_copy(edges_hbm.at[wid, k + 2], edge.at[b2],
                                      ses[b2]).wait()
                pltpu.make_async_copy(vals_hbm.at[wid, k + 2], ev.at[b2],
                                      ses[b2]).wait()
                pltpu.async_copy(h_hbm.at[edge.at[b2, 1]], gat.at[b2],
                                 sgs[b2])

            pltpu.make_async_copy(h_hbm.at[edge.at[b, 1]], gat.at[b],
                                  sgs[b]).wait()

            @plsc.parallel_loop(0, B // L, unroll=2)
            def _scale(g):
                v16 = ev[b, pl.ds(g * L, L)]
                for j in range(L):
                    vv = jnp.full((L,), v16[j], jnp.float32)
                    row = g * L + j
                    for h in range(D // L):
                        sl = pl.ds(h * L, L)
                        gat[b, row, sl] = gat[b, row, sl] * vv

            pltpu.async_copy(gat.at[b], acc.at[edge.at[b, 0]], sss[b],
                             add=True)

            @pl.when(k + NBUF < nt)
            def _():
                pltpu.async_copy(edges_hbm.at[wid, k + NBUF], edge.at[b],
                                 ses[b])
                pltpu.async_copy(vals_hbm.at[wid, k + NBUF], ev.at[b],
                                 ses[b])
        return 0

    lax.fori_loop(0, nt // NBUF, _quad, 0)
    pltpu.make_async_copy(gat.at[2], acc.at[edge.at[2, 0]], sss[2]).wait()
    pltpu.make_async_copy(gat.at[3], acc.at[edge.at[3, 0]], sss[3]).wait()
    plsc.subcore_barrier()

    pltpu.sync_copy(acc.at[pl.ds(s * STRIPE, STRIPE)],
                    out_hbm.at[c, pl.ds(s * STRIPE, STRIPE)])

    @pl.when(s == NS - 1)
    def _():
        pltpu.sync_copy(acc.at[pl.ds(NS * STRIPE, TAIL)],
                        out_hbm.at[c, pl.ds(NS * STRIPE, TAIL)])


def _spmm(edges, vals, H):
    nmax = max(N0, N1)
    mesh = plsc.VectorSubcoreMesh(core_axis_name="c", subcore_axis_name="s")
    f = pl.kernel(
        _spmm_body,
        out_type=jax.ShapeDtypeStruct((NC, C, D), jnp.float32),
        mesh=mesh,
        scratch_types=[
            pltpu.VMEM((NBUF, 2, B), jnp.int32),
            pltpu.VMEM((NBUF, B), jnp.float32),
            pltpu.VMEM((NBUF, B, D), jnp.float32),
            pltpu.VMEM_SHARED((C, D), jnp.float32),
        ] + [pltpu.SemaphoreType.DMA] * 12,
    )
    del nmax
    return f(edges, vals, H)


def _mm_body(p_ref, w_ref, o_ref, *, relu):
    x = p_ref[0] + p_ref[1]
    y = jnp.dot(x, w_ref[...], preferred_element_type=jnp.float32)
    if relu:
        y = jnp.maximum(y, 0.0)
    o_ref[...] = y


def _merge_matmul(P, W, relu):
    R = 2000
    return pl.pallas_call(
        functools.partial(_mm_body, relu=relu),
        grid=(C // R,),
        in_specs=[
            pl.BlockSpec((2, R, D), lambda i: (0, i, 0)),
            pl.BlockSpec((D, D), lambda i: (0, 0)),
        ],
        out_specs=pl.BlockSpec((R, D), lambda i: (i, 0)),
        out_shape=jax.ShapeDtypeStruct((C, D), jnp.float32),
    )(P, W)


def _pack_edges(arr, pad_value, dtype):
    nmax = max(N0, N1)
    e0 = NS * N0 * B
    p0 = arr[:e0].reshape(NS, N0, B)
    p1 = arr[e0:].reshape(NS, N1, B)
    fill0 = jnp.full((NS, nmax - N0, B), pad_value, dtype)
    fill1 = jnp.full((NS, nmax - N1, B), pad_value, dtype)
    return jnp.concatenate([
        jnp.concatenate([p0, fill0], axis=1),
        jnp.concatenate([p1, fill1], axis=1)], axis=0)


def kernel(H, A_hat_indices, A_hat_values, W1, W2):
    E = A_hat_values.shape[0]
    pe = NS * (N0 + N1) * B
    pad = pe - E
    rows = jnp.concatenate([A_hat_indices[0], jnp.zeros((pad,), jnp.int32)])
    cols = jnp.concatenate([A_hat_indices[1], jnp.zeros((pad,), jnp.int32)])
    vals = jnp.concatenate([A_hat_values, jnp.zeros((pad,), jnp.float32)])
    edges = jnp.stack([_pack_edges(rows, 0, jnp.int32),
                       _pack_edges(cols, 0, jnp.int32)], axis=2)
    vals = _pack_edges(vals, 0.0, jnp.float32)

    P1 = _spmm(edges, vals, H)
    H1 = _merge_matmul(P1, W1, relu=True)
    P2 = _spmm(edges, vals, H1)
    H2 = _merge_matmul(P2, W2, relu=False)
    return H2


# speedup vs baseline: 1.1306x; 1.1306x over previous
import functools

import jax
import jax.numpy as jnp
from jax import lax
from jax.experimental import pallas as pl
from jax.experimental.pallas import tpu as pltpu
from jax.experimental.pallas import tpu_sc as plsc

C = 10000
D = 128
L = 16
NC = 2
NS = 16
NW = NC * NS
B = 96
NBUF = 4
N0 = 132
N1 = 84
STRIPE = 624
TAIL = C - NS * STRIPE


def _spmm_body(edges_hbm, vals_hbm, h_hbm, out_hbm,
               edge, ev, gat, acc,
               sg0, sg1, sg2, sg3, ss0, ss1, ss2, ss3,
               se0, se1, se2, se3):
    c = lax.axis_index("c")
    s = lax.axis_index("s")
    wid = c * NS + s
    nt = jnp.where(c == 0, N0, N1)
    sgs = [sg0, sg1, sg2, sg3]
    sss = [ss0, ss1, ss2, ss3]
    ses = [se0, se1, se2, se3]

    for b in range(NBUF):
        pltpu.async_copy(edges_hbm.at[wid, b], edge.at[b], ses[b])
        pltpu.async_copy(vals_hbm.at[wid, b], ev.at[b], ses[b])

    zero = jnp.zeros((L,), jnp.float32)

    def _zero_row(i, _):
        for h in range(D // L):
            gat[0, i, pl.ds(h * L, L)] = zero
        return 0

    lax.fori_loop(0, B, _zero_row, 0)
    for k in range(6):
        pltpu.sync_copy(gat.at[0], acc.at[pl.ds(s * STRIPE + k * B, B)])
    pltpu.sync_copy(gat.at[0, pl.ds(0, 48)],
                    acc.at[pl.ds(s * STRIPE + 6 * B, 48)])

    @pl.when(s == NS - 1)
    def _():
        pltpu.sync_copy(gat.at[0, pl.ds(0, TAIL)],
                        acc.at[pl.ds(NS * STRIPE, TAIL)])

    plsc.subcore_barrier()

    for b in range(2):
        pltpu.make_async_copy(edges_hbm.at[wid, b], edge.at[b],
                              ses[b]).wait()
        pltpu.make_async_copy(vals_hbm.at[wid, b], ev.at[b], ses[b]).wait()
        pltpu.async_copy(h_hbm.at[edge.at[b, 1]], gat.at[b], sgs[b])

    def _quad(kk, _):
        for b in range(NBUF):
            k = kk * NBUF + b
            b2 = (b + 2) % NBUF

            @pl.when(k >= 2)
            def _():
                pltpu.make_async_copy(
                    gat.at[b2], acc.at[edge.at[b2, 0]], sss[b2]).wait()

            @pl.when(k + 2 < nt)
            def _():
                pltpu.make_async_copy(edges_hbm.at[wid, k + 2], edge.at[b2],
                                      ses[b2]).wait()
                pltpu.make_async_copy(vals_hbm.at[wid, k + 2], ev.at[b2],
                                      ses[b2]).wait()
                pltpu.async_copy(h_hbm.at[edge.at[b2, 1]], gat.at[b2],
                                 sgs[b2])

            pltpu.make_async_copy(h_hbm.at[edge.at[b, 1]], gat.at[b],
                                  sgs[b]).wait()

            @plsc.parallel_loop(0, B // L, unroll=2)
            def _scale(g):
                v16 = ev[b, pl.ds(g * L, L)]
                for j in range(L):
                    vv = jnp.full((L,), v16[j], jnp.float32)
                    row = g * L + j
                    for h in range(D // L):
                        sl = pl.ds(h * L, L)
                        gat[b, row, sl] = gat[b, row, sl] * vv

            pltpu.async_copy(gat.at[b], acc.at[edge.at[b, 0]], sss[b],
                             add=True)

            @pl.when(k + NBUF < nt)
            def _():
                pltpu.async_copy(edges_hbm.at[wid, k + NBUF], edge.at[b],
                                 ses[b])
                pltpu.async_copy(vals_hbm.at[wid, k + NBUF], ev.at[b],
                                 ses[b])
        return 0

    lax.fori_loop(0, nt // NBUF, _quad, 0)
    pltpu.make_async_copy(gat.at[2], acc.at[edge.at[2, 0]], sss[2]).wait()
    pltpu.make_async_copy(gat.at[3], acc.at[edge.at[3, 0]], sss[3]).wait()
    plsc.subcore_barrier()

    pltpu.sync_copy(acc.at[pl.ds(s * STRIPE, STRIPE)],
                    out_hbm.at[c, pl.ds(s * STRIPE, STRIPE)])

    @pl.when(s == NS - 1)
    def _():
        pltpu.sync_copy(acc.at[pl.ds(NS * STRIPE, TAIL)],
                        out_hbm.at[c, pl.ds(NS * STRIPE, TAIL)])


def _spmm(edges, vals, H):
    nmax = max(N0, N1)
    mesh = plsc.VectorSubcoreMesh(core_axis_name="c", subcore_axis_name="s")
    f = pl.kernel(
        _spmm_body,
        out_type=jax.ShapeDtypeStruct((NC, C, D), jnp.float32),
        mesh=mesh,
        scratch_types=[
            pltpu.VMEM((NBUF, 2, B), jnp.int32),
            pltpu.VMEM((NBUF, B), jnp.float32),
            pltpu.VMEM((NBUF, B, D), jnp.float32),
            pltpu.VMEM_SHARED((C, D), jnp.float32),
        ] + [pltpu.SemaphoreType.DMA] * 12,
    )
    del nmax
    return f(edges, vals, H)


def _mm_body(p_ref, w_ref, o_ref, *, relu):
    x = p_ref[0] + p_ref[1]
    y = jnp.dot(x, w_ref[...], preferred_element_type=jnp.float32)
    if relu:
        y = jnp.maximum(y, 0.0)
    o_ref[...] = y


def _merge_matmul(P, W, relu):
    R = 2000
    return pl.pallas_call(
        functools.partial(_mm_body, relu=relu),
        grid=(C // R,),
        in_specs=[
            pl.BlockSpec((2, R, D), lambda i: (0, i, 0)),
            pl.BlockSpec((D, D), lambda i: (0, 0)),
        ],
        out_specs=pl.BlockSpec((R, D), lambda i: (i, 0)),
        out_shape=jax.ShapeDtypeStruct((C, D), jnp.float32),
    )(P, W)


def _pack_edges(arr, pad_value, dtype):
    nmax = max(N0, N1)
    e0 = NS * N0 * B
    p0 = arr[:e0].reshape(NS, N0, B)
    p1 = arr[e0:].reshape(NS, N1, B)
    fill0 = jnp.full((NS, nmax - N0, B), pad_value, dtype)
    fill1 = jnp.full((NS, nmax - N1, B), pad_value, dtype)
    return jnp.concatenate([
        jnp.concatenate([p0, fill0], axis=1),
        jnp.concatenate([p1, fill1], axis=1)], axis=0)


def kernel(H, A_hat_indices, A_hat_values, W1, W2):
    E = A_hat_values.shape[0]
    pe = NS * (N0 + N1) * B
    pad = pe - E
    rows = jnp.concatenate([A_hat_indices[0], jnp.zeros((pad,), jnp.int32)])
    cols = jnp.concatenate([A_hat_indices[1], jnp.zeros((pad,), jnp.int32)])
    vals = jnp.concatenate([A_hat_values, jnp.zeros((pad,), jnp.float32)])
    edges = jnp.stack([_pack_edges(rows, 0, jnp.int32),
                       _pack_edges(cols, 0, jnp.int32)], axis=2)
    vals = _pack_edges(vals, 0.0, jnp.float32)

    P1 = _spmm(edges, vals, H)
    H1 = _merge_matmul(P1, W1, relu=True)
    P2 = _spmm(edges, vals, H1)
    H2 = _merge_matmul(P2, W2, relu=False)
    return H2
